# baseline (speedup 1.0000x reference)
import functools

import jax
import jax.numpy as jnp
from jax import lax
from jax.experimental import pallas as pl
from jax.experimental.pallas import tpu as pltpu
from jax.experimental.pallas import tpu_sc as plsc

_B, _C, _NUM_BANDS = 32, 256, 64
_NC, _NS = 2, 16
_GATHER_WORKERS = 4
_ROWS_PER_W = _B // _GATHER_WORKERS


def _sc_gather_body(table_hbm, idx_hbm, gamma_hbm, beta_hbm, idx_v, rows_v, sem):
    wid = lax.axis_index("s") * _NC + lax.axis_index("c")

    @pl.when(wid < _GATHER_WORKERS)
    def _():
        base = wid * _ROWS_PER_W
        pltpu.sync_copy(idx_hbm.at[pl.ds(base, _ROWS_PER_W)], idx_v)
        pltpu.async_copy(table_hbm.at[idx_v], rows_v, sem).wait()
        pltpu.sync_copy(rows_v.at[:, pl.ds(0, _C)], gamma_hbm.at[pl.ds(base, _ROWS_PER_W)])
        pltpu.sync_copy(rows_v.at[:, pl.ds(_C, _C)], beta_hbm.at[pl.ds(base, _ROWS_PER_W)])


@jax.jit
def _sc_gather(embed_weight, idx):
    mesh = plsc.VectorSubcoreMesh(core_axis_name="c", subcore_axis_name="s")
    return pl.kernel(
        _sc_gather_body,
        out_type=(
            jax.ShapeDtypeStruct((_B, _C), jnp.float32),
            jax.ShapeDtypeStruct((_B, _C), jnp.float32),
        ),
        mesh=mesh,
        scratch_types=[
            pltpu.VMEM((_ROWS_PER_W,), jnp.int32),
            pltpu.VMEM((_ROWS_PER_W, 2 * _C), jnp.float32),
            pltpu.SemaphoreType.DMA,
        ],
    )(embed_weight, idx)


def _film_body(gamma_ref, beta_ref, x_ref, o_ref):
    g = gamma_ref[0]
    b = beta_ref[0]
    o_ref[0] = x_ref[0] * (1.0 + g) + b


def _film(gamma_col, beta_col, x3, c_blk):
    B, C, HW = x3.shape
    grid = (B, C // c_blk)
    gb_spec = pl.BlockSpec((1, c_blk, 1), lambda b, c: (b, c, 0))
    x_spec = pl.BlockSpec((1, c_blk, HW), lambda b, c: (b, c, 0))
    return pl.pallas_call(
        _film_body,
        grid=grid,
        in_specs=[gb_spec, gb_spec, x_spec],
        out_specs=x_spec,
        out_shape=jax.ShapeDtypeStruct(x3.shape, x3.dtype),
        compiler_params=pltpu.CompilerParams(
            dimension_semantics=("parallel", "parallel"),
        ),
    )(gamma_col, beta_col, x3)


def kernel(x, band_idx, embed_weight):
    B, C, H, W = x.shape
    idx = band_idx.astype(jnp.int32)
    gamma, beta = _sc_gather(embed_weight, idx)
    out = _film(
        gamma.reshape(B, C, 1),
        beta.reshape(B, C, 1),
        x.reshape(B, C, H * W),
        c_blk=128,
    )
    return out.reshape(B, C, H, W)
